# Initial kernel scaffold; baseline (speedup 1.0000x reference)
import jax
import jax.numpy as jnp
from jax.experimental import pallas as pl


def kernel(x, weight, bias):
    raise NotImplementedError("write your pallas kernel here")


# speedup vs baseline: 1.5317x; 1.5317x over previous
import functools

import jax
import jax.numpy as jnp
from jax.experimental import pallas as pl

DIM = 4096
N_EXPERTS = 64
TOPK = 8
N_GROUPS = 8
GROUP_SIZE = N_EXPERTS // N_GROUPS
TOPK_GROUPS = 4
ROUTE_SCALE = 2.5

NEG_INF = float("-inf")


def _gate_kernel(x_ref, wt_ref, bias_ref, w_out_ref, idx_out_ref):
    x = x_ref[...]
    wt = wt_ref[...]
    scores = jnp.dot(x, wt, preferred_element_type=jnp.float32)
    s = jax.nn.sigmoid(scores)
    sb = s + bias_ref[...]
    bt = s.shape[0]

    giota = jax.lax.broadcasted_iota(jnp.int32, (bt, GROUP_SIZE), 1)
    gscores = []
    for g in range(N_GROUPS):
        blk = sb[:, g * GROUP_SIZE:(g + 1) * GROUP_SIZE]
        m1 = jnp.max(blk, axis=1, keepdims=True)
        pos1 = jnp.min(jnp.where(blk == m1, giota, GROUP_SIZE),
                       axis=1, keepdims=True)
        m2 = jnp.max(jnp.where(giota == pos1, NEG_INF, blk),
                     axis=1, keepdims=True)
        gscores.append(m1 + m2)
    group_scores = jnp.concatenate(gscores, axis=1)

    grp_iota = jax.lax.broadcasted_iota(jnp.int32, (bt, N_GROUPS), 1)
    lane_iota = jax.lax.broadcasted_iota(jnp.int32, (bt, N_EXPERTS), 1)
    lane_grp = lane_iota // GROUP_SIZE
    allowed = jnp.zeros((bt, N_EXPERTS), dtype=jnp.bool_)
    cur = group_scores
    for _ in range(TOPK_GROUPS):
        m = jnp.max(cur, axis=1, keepdims=True)
        pos = jnp.min(jnp.where(cur == m, grp_iota, N_GROUPS),
                      axis=1, keepdims=True)
        allowed = allowed | (lane_grp == pos)
        cur = jnp.where(grp_iota == pos, NEG_INF, cur)

    masked = jnp.where(allowed, sb, NEG_INF)
    w_cols = []
    idx_cols = []
    for _ in range(TOPK):
        m = jnp.max(masked, axis=1, keepdims=True)
        pos = jnp.min(jnp.where(masked == m, lane_iota, N_EXPERTS),
                      axis=1, keepdims=True)
        hit = lane_iota == pos
        w_cols.append(jnp.max(jnp.where(hit, s, NEG_INF), axis=1,
                              keepdims=True))
        idx_cols.append(pos)
        masked = jnp.where(hit, NEG_INF, masked)

    topw = jnp.concatenate(w_cols, axis=1)
    topw = topw / jnp.sum(topw, axis=1, keepdims=True) * ROUTE_SCALE
    w_out_ref[...] = topw
    idx_out_ref[...] = jnp.concatenate(idx_cols, axis=1)


@functools.partial(jax.jit, static_argnames=())
def kernel(x, weight, bias):
    n_tok = x.shape[0]
    bt = 512
    grid = (n_tok // bt,)
    wt = weight.T
    bias2 = bias.reshape(1, N_EXPERTS)
    w_out, idx_out = pl.pallas_call(
        _gate_kernel,
        grid=grid,
        in_specs=[
            pl.BlockSpec((bt, DIM), lambda i: (i, 0)),
            pl.BlockSpec((DIM, N_EXPERTS), lambda i: (0, 0)),
            pl.BlockSpec((1, N_EXPERTS), lambda i: (0, 0)),
        ],
        out_specs=[
            pl.BlockSpec((bt, TOPK), lambda i: (i, 0)),
            pl.BlockSpec((bt, TOPK), lambda i: (i, 0)),
        ],
        out_shape=[
            jax.ShapeDtypeStruct((n_tok, TOPK), jnp.float32),
            jax.ShapeDtypeStruct((n_tok, TOPK), jnp.int32),
        ],
    )(x, wt, bias2)
    return w_out.astype(x.dtype), idx_out


# speedup vs baseline: 3.4647x; 2.2620x over previous
import functools

import jax
import jax.numpy as jnp
from jax import lax
from jax.experimental import pallas as pl
from jax.experimental.pallas import tpu as pltpu
from jax.experimental.pallas import tpu_sc as plsc

DIM = 4096
N_EXPERTS = 64
TOPK = 8
N_GROUPS = 8
GROUP_SIZE = N_EXPERTS // N_GROUPS
TOPK_GROUPS = 4
ROUTE_SCALE = 2.5

NC = 2
NS = 16
L = 16
NW = NC * NS

NEG_INF = float("-inf")


def _score_kernel(x_ref, wt_ref, out_ref):
    scores = jnp.dot(x_ref[...], wt_ref[...],
                     preferred_element_type=jnp.float32)
    out_ref[...] = jax.nn.sigmoid(scores).T


def _route_body(s_hbm, biasb_hbm, w_hbm, idx_hbm, s_v, biasb_v, w_v, idx_v):
    tpw = s_v.shape[1]
    wid = lax.axis_index("s") * NC + lax.axis_index("c")
    base = wid * tpw
    pltpu.sync_copy(s_hbm.at[:, pl.ds(base, tpw)], s_v)
    pltpu.sync_copy(biasb_hbm, biasb_v)

    neg = jnp.full((L,), NEG_INF, jnp.float32)
    zero_i = jnp.full((L,), 0, jnp.int32)
    one_i = jnp.full((L,), 1, jnp.int32)

    def chunk(c, carry):
        off = c * L

        def sb_row(e):
            return s_v[e, pl.ds(off, L)] + biasb_v[e, :]

        gs = []
        for g in range(N_GROUPS):
            m1 = sb_row(g * GROUP_SIZE)
            m2 = neg
            for j in range(1, GROUP_SIZE):
                x = sb_row(g * GROUP_SIZE + j)
                c1 = x > m1
                c2 = x > m2
                m2 = jnp.where(c1, m1, jnp.where(c2, x, m2))
                m1 = jnp.where(c1, x, m1)
            gs.append(m1 + m2)

        t = [neg] * TOPK_GROUPS
        for g in range(N_GROUPS):
            x = gs[g]
            b = [x > tk for tk in t]
            nt = [jnp.where(b[0], x, t[0])]
            for k in range(1, TOPK_GROUPS):
                nt.append(jnp.where(b[k], jnp.where(b[k - 1], t[k - 1], x),
                                    t[k]))
            t = nt
        t4 = t[TOPK_GROUPS - 1]

        cnt_gt = zero_i
        gtv = []
        for g in range(N_GROUPS):
            gt = gs[g] > t4
            gtv.append(gt)
            cnt_gt = cnt_gt + jnp.where(gt, one_i, zero_i)
        need = jnp.full((L,), TOPK_GROUPS, jnp.int32) - cnt_gt
        eqr = zero_i
        allowed = []
        for g in range(N_GROUPS):
            eq = gs[g] == t4
            allowed.append(gtv[g] | (eq & (eqr < need)))
            eqr = eqr + jnp.where(eq, one_i, zero_i)

        sv = [neg] * TOPK
        si = [zero_i] * TOPK
        for g in range(N_GROUPS):
            for j in range(GROUP_SIZE):
                e = g * GROUP_SIZE + j
                x = jnp.where(allowed[g], sb_row(e), neg)
                e_c = jnp.full((L,), e, jnp.int32)
                b = [x > v for v in sv]
                nv = [jnp.where(b[0], x, sv[0])]
                ni = [jnp.where(b[0], e_c, si[0])]
                for k in range(1, TOPK):
                    nv.append(jnp.where(b[k],
                                        jnp.where(b[k - 1], sv[k - 1], x),
                                        sv[k]))
                    ni.append(jnp.where(b[k],
                                        jnp.where(b[k - 1], si[k - 1], e_c),
                                        si[k]))
                sv, si = nv, ni

        tok = jnp.arange(L, dtype=jnp.int32) + off
        ws = [plsc.load_gather(s_v, [si[k], tok]) for k in range(TOPK)]
        total = ws[0]
        for k in range(1, TOPK):
            total = total + ws[k]
        scale = jnp.full((L,), ROUTE_SCALE, jnp.float32) / total
        for k in range(TOPK):
            w_v[k, pl.ds(off, L)] = ws[k] * scale
            idx_v[k, pl.ds(off, L)] = si[k]
        return carry

    lax.fori_loop(0, tpw // L, chunk, 0)

    pltpu.sync_copy(w_v, w_hbm.at[:, pl.ds(base, tpw)])
    pltpu.sync_copy(idx_v, idx_hbm.at[:, pl.ds(base, tpw)])


@jax.jit
def kernel(x, weight, bias):
    n_tok = x.shape[0]
    bt = 512
    s_t = pl.pallas_call(
        _score_kernel,
        grid=(n_tok // bt,),
        in_specs=[
            pl.BlockSpec((bt, DIM), lambda i: (i, 0)),
            pl.BlockSpec((DIM, N_EXPERTS), lambda i: (0, 0)),
        ],
        out_specs=pl.BlockSpec((N_EXPERTS, bt), lambda i: (0, i)),
        out_shape=jax.ShapeDtypeStruct((N_EXPERTS, n_tok), jnp.float32),
    )(x, weight.T)

    tpw = n_tok // NW
    biasb = jnp.broadcast_to(bias[:, None], (N_EXPERTS, L))
    route = pl.kernel(
        _route_body,
        out_type=[
            jax.ShapeDtypeStruct((TOPK, n_tok), jnp.float32),
            jax.ShapeDtypeStruct((TOPK, n_tok), jnp.int32),
        ],
        mesh=plsc.VectorSubcoreMesh(core_axis_name="c", subcore_axis_name="s"),
        scratch_types=[
            pltpu.VMEM((N_EXPERTS, tpw), jnp.float32),
            pltpu.VMEM((N_EXPERTS, L), jnp.float32),
            pltpu.VMEM((TOPK, tpw), jnp.float32),
            pltpu.VMEM((TOPK, tpw), jnp.int32),
        ],
        compiler_params=pltpu.CompilerParams(use_tc_tiling_on_sc=False,
                                             needs_layout_passes=False),
    )
    w_t, idx_t = route(s_t, biasb)
    return w_t.T.astype(x.dtype), idx_t.T
